# baseline (speedup 1.0000x reference)
import functools

import jax
import jax.numpy as jnp
from jax import lax
from jax.experimental import pallas as pl
from jax.experimental.pallas import tpu as pltpu
from jax.experimental.pallas import tpu_sc as plsc

BOS = 0
CHUNK = 32
LANES = 16


@functools.lru_cache(maxsize=None)
def _build(batch: int, seq: int, vocab: int, d_model: int):
    info = plsc.get_sparse_core_info()
    nc = info.num_cores
    num_workers = nc * info.num_subcores
    assert vocab % 8 == 0 and d_model % LANES == 0

    out_t = seq + 1
    wpb = num_workers // batch
    per_w = seq // wpb
    assert per_w % CHUNK == 0 and per_w * wpb == seq
    n_chunks = per_w // CHUNK
    row_p = seq + 8

    mesh = plsc.VectorSubcoreMesh(core_axis_name="c", subcore_axis_name="s")

    @functools.partial(
        pl.kernel,
        mesh=mesh,
        out_type=jax.ShapeDtypeStruct((batch, out_t, d_model), jnp.float32),
        scratch_types=[
            pltpu.VMEM((per_w + LANES,), jnp.int32),
            pltpu.VMEM((CHUNK, 8, d_model), jnp.float32),
            pltpu.VMEM((CHUNK, 8, d_model), jnp.float32),
            pltpu.VMEM((CHUNK, d_model), jnp.float32),
            pltpu.VMEM((LANES, 8, d_model), jnp.float32),
            pltpu.VMEM((LANES, d_model), jnp.float32),
            pltpu.VMEM((2 * LANES,), jnp.int32),
            pltpu.SemaphoreType.DMA,
            pltpu.SemaphoreType.DMA,
            pltpu.SemaphoreType.DMA,
        ],
        compiler_params=pltpu.CompilerParams(needs_layout_passes=False),
    )
    def emb(ids_hbm, table_hbm, out_hbm, idx_v, tiles_a, tiles_b,
            rows_v, ttiles_v, trows_v, tidx_v, sem_a, sem_b, sem_t):
        w = lax.axis_index("s") * nc + lax.axis_index("c")
        b = w // wpb
        sub = w % wpb
        ids_base = pl.multiple_of(b * row_p + sub * per_w, 8)
        view3 = table_hbm.reshape(vocab // 8, 8, d_model)

        pltpu.sync_copy(ids_hbm.at[pl.ds(ids_base, per_w)],
                        idx_v.at[pl.ds(0, per_w)])
        lane0 = lax.iota(jnp.int32, LANES) == jnp.full((LANES,), 0, jnp.int32)
        zeros = jnp.full((LANES,), 0, jnp.int32)

        def scalar_at(ref, j):
            win = ref[pl.ds(j, LANES)]
            return lax.reduce_max(jnp.where(lane0, win, zeros), axes=(0,))

        bufs = (tiles_a, tiles_b)
        sems = (sem_a, sem_b)

        def fire(c):
            buf, sem = bufs[c % 2], sems[c % 2]

            @plsc.parallel_loop(0, CHUNK, unroll=8)
            def _f(j):
                tid = c * CHUNK + j
                pltpu.async_copy(view3.at[tid], buf.at[j], sem)

        def drain(c):
            buf, sem = bufs[c % 2], sems[c % 2]
            pltpu.make_async_copy(view3.at[pl.ds(0, CHUNK)], buf, sem).wait()

        fire(0)
        for c in range(n_chunks):
            if c + 1 < n_chunks:
                fire(c + 1)
            drain(c)
            buf = bufs[c % 2]
            grp = []
            for g in range(CHUNK // LANES):
                tokjv = jnp.full((LANES,), g * LANES, jnp.int32) + \
                    lax.iota(jnp.int32, LANES)
                subv = jnp.bitwise_and(
                    idx_v[pl.ds(c * CHUNK + g * LANES, LANES)],
                    jnp.full((LANES,), 7, jnp.int32))
                grp.append((tokjv, subv))

            @plsc.parallel_loop(0, d_model, unroll=4)
            def _e(col):
                colv = jnp.full((LANES,), col, jnp.int32)
                for tokjv, subv in grp:
                    v = plsc.load_gather(buf, [tokjv, subv, colv])
                    plsc.store_scatter(rows_v, [tokjv, colv], v)

            pltpu.sync_copy(
                rows_v,
                out_hbm.at[b, pl.ds(
                    pl.multiple_of(sub * per_w + c * CHUNK, 8), CHUNK)])

        @pl.when(w < batch)
        def _tail():
            tb = w
            t_base = pl.multiple_of(tb * row_p + seq - 8, 8)
            pltpu.sync_copy(ids_hbm.at[pl.ds(t_base, LANES)],
                            tidx_v.at[pl.ds(0, LANES)])

            def tbody(j, _):
                tid = lax.shift_right_logical(scalar_at(tidx_v, j), 3)
                pltpu.async_copy(view3.at[tid], ttiles_v.at[j], sem_t)
                return _

            lax.fori_loop(0, LANES, tbody, 0)
            pltpu.make_async_copy(
                view3.at[pl.ds(0, LANES)], ttiles_v, sem_t).wait()
            tokv = lax.iota(jnp.int32, LANES)
            subv = jnp.bitwise_and(
                tidx_v[pl.ds(0, LANES)], jnp.full((LANES,), 7, jnp.int32))

            def textract(col, _):
                colv = jnp.full((LANES,), col, jnp.int32)
                v = plsc.load_gather(ttiles_v, [tokv, subv, colv])
                plsc.store_scatter(trows_v, [tokv, colv], v)
                return _

            lax.fori_loop(0, d_model, textract, 0)
            pltpu.sync_copy(trows_v.at[pl.ds(8, 1)],
                            out_hbm.at[tb, pl.ds(seq, 1)])

    return emb


def kernel(input_ids, embedding_table):
    batch, seq = input_ids.shape
    vocab, d_model = embedding_table.shape
    padded = jnp.pad(input_ids, ((0, 0), (1, 7)), constant_values=BOS)
    flat_ids = padded.reshape(-1).astype(jnp.int32)
    emb = _build(batch, seq, vocab, d_model)
    return emb(flat_ids, embedding_table)


# speedup vs baseline: 1.1620x; 1.1620x over previous
import functools

import jax
import jax.numpy as jnp
from jax import lax
from jax.experimental import pallas as pl
from jax.experimental.pallas import tpu as pltpu
from jax.experimental.pallas import tpu_sc as plsc

BOS = 0
CHUNK = 32
LANES = 16


@functools.lru_cache(maxsize=None)
def _build(batch: int, seq: int, vocab: int, d_model: int):
    info = plsc.get_sparse_core_info()
    nc = info.num_cores
    num_workers = nc * info.num_subcores
    assert vocab % 8 == 0 and d_model % LANES == 0

    out_t = seq + 1
    wpb = num_workers // batch
    per_w = seq // wpb
    assert per_w % CHUNK == 0 and per_w * wpb == seq
    n_chunks = per_w // CHUNK
    row_p = seq + 8

    mesh = plsc.VectorSubcoreMesh(core_axis_name="c", subcore_axis_name="s")

    @functools.partial(
        pl.kernel,
        mesh=mesh,
        out_type=jax.ShapeDtypeStruct((batch, out_t, d_model), jnp.float32),
        scratch_types=[
            pltpu.VMEM((per_w + LANES,), jnp.int32),
            pltpu.VMEM((CHUNK, 8, d_model), jnp.float32),
            pltpu.VMEM((CHUNK, 8, d_model), jnp.float32),
            pltpu.VMEM((CHUNK, d_model), jnp.float32),
            pltpu.VMEM((CHUNK, d_model), jnp.float32),
            pltpu.VMEM((LANES, 8, d_model), jnp.float32),
            pltpu.VMEM((LANES, d_model), jnp.float32),
            pltpu.VMEM((2 * LANES,), jnp.int32),
            pltpu.SemaphoreType.DMA,
            pltpu.SemaphoreType.DMA,
            pltpu.SemaphoreType.DMA,
            pltpu.SemaphoreType.DMA,
            pltpu.SemaphoreType.DMA,
        ],
        compiler_params=pltpu.CompilerParams(needs_layout_passes=False),
    )
    def emb(ids_hbm, table_hbm, out_hbm, idx_v, tiles_a, tiles_b,
            rows_a, rows_b, ttiles_v, trows_v, tidx_v,
            sem_a, sem_b, sem_t, sem_oa, sem_ob):
        w = lax.axis_index("s") * nc + lax.axis_index("c")
        b = w // wpb
        sub = w % wpb
        ids_base = pl.multiple_of(b * row_p + sub * per_w, 8)
        view3 = table_hbm.reshape(vocab // 8, 8, d_model)

        pltpu.sync_copy(ids_hbm.at[pl.ds(ids_base, per_w)],
                        idx_v.at[pl.ds(0, per_w)])
        lane0 = lax.iota(jnp.int32, LANES) == jnp.full((LANES,), 0, jnp.int32)
        zeros = jnp.full((LANES,), 0, jnp.int32)

        def scalar_at(ref, j):
            win = ref[pl.ds(j, LANES)]
            return lax.reduce_max(jnp.where(lane0, win, zeros), axes=(0,))

        bufs = (tiles_a, tiles_b)
        sems = (sem_a, sem_b)

        def fire(c):
            buf, sem = bufs[c % 2], sems[c % 2]

            @plsc.parallel_loop(0, CHUNK, unroll=8)
            def _f(j):
                tid = lax.shift_right_logical(
                    scalar_at(idx_v, c * CHUNK + j), 3)
                pltpu.async_copy(view3.at[tid], buf.at[j], sem)

        def drain(c):
            buf, sem = bufs[c % 2], sems[c % 2]
            pltpu.make_async_copy(view3.at[pl.ds(0, CHUNK)], buf, sem).wait()

        rows_bufs = (rows_a, rows_b)
        out_sems = (sem_oa, sem_ob)
        out_cp = [None, None]
        fire(0)
        for c in range(n_chunks):
            if c + 1 < n_chunks:
                fire(c + 1)
            drain(c)
            par = c % 2
            buf = bufs[par]
            rows_v = rows_bufs[par]
            if out_cp[par] is not None:
                out_cp[par].wait()
            grp = []
            for g in range(CHUNK // LANES):
                tokjv = jnp.full((LANES,), g * LANES, jnp.int32) + \
                    lax.iota(jnp.int32, LANES)
                subv = jnp.bitwise_and(
                    idx_v[pl.ds(c * CHUNK + g * LANES, LANES)],
                    jnp.full((LANES,), 7, jnp.int32))
                grp.append((tokjv, subv))

            @plsc.parallel_loop(0, d_model, unroll=4)
            def _e(col):
                colv = jnp.full((LANES,), col, jnp.int32)
                for tokjv, subv in grp:
                    v = plsc.load_gather(buf, [tokjv, subv, colv])
                    plsc.store_scatter(rows_v, [tokjv, colv], v)

            out_cp[par] = pltpu.async_copy(
                rows_v,
                out_hbm.at[b, pl.ds(
                    pl.multiple_of(sub * per_w + c * CHUNK, 8), CHUNK)],
                out_sems[par])

        for cp in out_cp:
            cp.wait()

        @pl.when(w < batch)
        def _tail():
            tb = w
            t_base = pl.multiple_of(tb * row_p + seq - 8, 8)
            pltpu.sync_copy(ids_hbm.at[pl.ds(t_base, LANES)],
                            tidx_v.at[pl.ds(0, LANES)])

            def tbody(j, _):
                tid = lax.shift_right_logical(scalar_at(tidx_v, j), 3)
                pltpu.async_copy(view3.at[tid], ttiles_v.at[j], sem_t)
                return _

            lax.fori_loop(0, LANES, tbody, 0)
            pltpu.make_async_copy(
                view3.at[pl.ds(0, LANES)], ttiles_v, sem_t).wait()
            tokv = lax.iota(jnp.int32, LANES)
            subv = jnp.bitwise_and(
                tidx_v[pl.ds(0, LANES)], jnp.full((LANES,), 7, jnp.int32))

            def textract(col, _):
                colv = jnp.full((LANES,), col, jnp.int32)
                v = plsc.load_gather(ttiles_v, [tokv, subv, colv])
                plsc.store_scatter(trows_v, [tokv, colv], v)
                return _

            lax.fori_loop(0, d_model, textract, 0)
            pltpu.sync_copy(trows_v.at[pl.ds(8, 1)],
                            out_hbm.at[tb, pl.ds(seq, 1)])

    return emb


def kernel(input_ids, embedding_table):
    batch, seq = input_ids.shape
    vocab, d_model = embedding_table.shape
    padded = jnp.pad(input_ids, ((0, 0), (1, 7)), constant_values=BOS)
    flat_ids = padded.reshape(-1).astype(jnp.int32)
    emb = _build(batch, seq, vocab, d_model)
    return emb(flat_ids, embedding_table)


# speedup vs baseline: 1.3032x; 1.1215x over previous
import functools

import jax
import jax.numpy as jnp
from jax import lax
from jax.experimental import pallas as pl
from jax.experimental.pallas import tpu as pltpu
from jax.experimental.pallas import tpu_sc as plsc

BOS = 0
CHUNK = 32
OUT_BLK = 128
LANES = 16


@functools.lru_cache(maxsize=None)
def _build(batch: int, seq: int, vocab: int, d_model: int):
    info = plsc.get_sparse_core_info()
    nc = info.num_cores
    num_workers = nc * info.num_subcores
    assert vocab % 8 == 0 and d_model % LANES == 0

    out_t = seq + 1
    wpb = num_workers // batch
    per_w = seq // wpb
    assert per_w % OUT_BLK == 0 and OUT_BLK % CHUNK == 0
    n_chunks = per_w // CHUNK
    cpb = OUT_BLK // CHUNK
    row_p = seq + 8

    mesh = plsc.VectorSubcoreMesh(core_axis_name="c", subcore_axis_name="s")

    @functools.partial(
        pl.kernel,
        mesh=mesh,
        out_type=(
            jax.ShapeDtypeStruct((batch, d_model, out_t), jnp.float32),
            jax.ShapeDtypeStruct((batch, d_model, LANES), jnp.float32),
        ),
        scratch_types=[
            pltpu.VMEM((per_w + LANES,), jnp.int32),
            pltpu.VMEM((CHUNK, 8, d_model), jnp.float32),
            pltpu.VMEM((CHUNK, 8, d_model), jnp.float32),
            pltpu.VMEM((d_model, OUT_BLK), jnp.float32),
            pltpu.VMEM((d_model, OUT_BLK), jnp.float32),
            pltpu.VMEM((LANES, 8, d_model), jnp.float32),
            pltpu.VMEM((d_model, LANES), jnp.float32),
            pltpu.VMEM((2 * LANES,), jnp.int32),
            pltpu.SemaphoreType.DMA,
            pltpu.SemaphoreType.DMA,
            pltpu.SemaphoreType.DMA,
            pltpu.SemaphoreType.DMA,
            pltpu.SemaphoreType.DMA,
        ],
        compiler_params=pltpu.CompilerParams(needs_layout_passes=False),
    )
    def emb(ids_hbm, table_hbm, out_hbm, tail_hbm, idx_v, tiles_a, tiles_b,
            rowsT_a, rowsT_b, ttiles_v, trowsT_v, tidx_v,
            sem_a, sem_b, sem_t, sem_oa, sem_ob):
        w = lax.axis_index("s") * nc + lax.axis_index("c")
        b = w // wpb
        sub = w % wpb
        ids_base = pl.multiple_of(b * row_p + sub * per_w, 8)
        view3 = table_hbm.reshape(vocab // 8, 8, d_model)

        pltpu.sync_copy(ids_hbm.at[pl.ds(ids_base, per_w)],
                        idx_v.at[pl.ds(0, per_w)])
        lane0 = lax.iota(jnp.int32, LANES) == jnp.full((LANES,), 0, jnp.int32)
        zeros = jnp.full((LANES,), 0, jnp.int32)

        def scalar_at(ref, j):
            win = ref[pl.ds(j, LANES)]
            return lax.reduce_max(jnp.where(lane0, win, zeros), axes=(0,))

        bufs = (tiles_a, tiles_b)
        sems = (sem_a, sem_b)

        def fire(c):
            buf, sem = bufs[c % 2], sems[c % 2]

            @plsc.parallel_loop(0, CHUNK, unroll=8)
            def _f(j):
                tid = lax.shift_right_logical(
                    scalar_at(idx_v, c * CHUNK + j), 3)
                pltpu.async_copy(view3.at[tid], buf.at[j], sem)

        def drain(c):
            buf, sem = bufs[c % 2], sems[c % 2]
            pltpu.make_async_copy(view3.at[pl.ds(0, CHUNK)], buf, sem).wait()

        rowsT_bufs = (rowsT_a, rowsT_b)
        out_sems = (sem_oa, sem_ob)
        out_cp = [None, None]
        fire(0)
        for c in range(n_chunks):
            if c + 1 < n_chunks:
                fire(c + 1)
            drain(c)
            buf = bufs[c % 2]
            blk = c // cpb
            bpar = blk % 2
            rowsT_v = rowsT_bufs[bpar]
            if c % cpb == 0 and out_cp[bpar] is not None:
                out_cp[bpar].wait()
            lblk = (c % cpb) * CHUNK
            grp = []
            for g in range(CHUNK // LANES):
                tokjv = jnp.full((LANES,), g * LANES, jnp.int32) + \
                    lax.iota(jnp.int32, LANES)
                subv = jnp.bitwise_and(
                    idx_v[pl.ds(c * CHUNK + g * LANES, LANES)],
                    jnp.full((LANES,), 7, jnp.int32))
                lanev = jnp.full((LANES,), lblk + g * LANES, jnp.int32) + \
                    lax.iota(jnp.int32, LANES)
                grp.append((tokjv, subv, lanev))

            @plsc.parallel_loop(0, d_model, unroll=4)
            def _e(col):
                colv = jnp.full((LANES,), col, jnp.int32)
                for tokjv, subv, lanev in grp:
                    v = plsc.load_gather(buf, [tokjv, subv, colv])
                    plsc.store_scatter(rowsT_v, [colv, lanev], v)

            if (c + 1) % cpb == 0:
                t0 = sub * per_w + blk * OUT_BLK
                out_cp[bpar] = pltpu.async_copy(
                    rowsT_v,
                    out_hbm.at[b, :, pl.ds(pl.multiple_of(t0, OUT_BLK),
                                           OUT_BLK)],
                    out_sems[bpar])

        for cp in out_cp:
            if cp is not None:
                cp.wait()

        @pl.when(sub == wpb - 1)
        def _tail():
            t_base = pl.multiple_of(b * row_p + seq - 8, 8)
            pltpu.sync_copy(ids_hbm.at[pl.ds(t_base, LANES)],
                            tidx_v.at[pl.ds(0, LANES)])

            def tbody(j, _):
                tid = lax.shift_right_logical(scalar_at(tidx_v, j), 3)
                pltpu.async_copy(view3.at[tid], ttiles_v.at[j], sem_t)
                return _

            lax.fori_loop(0, LANES, tbody, 0)
            pltpu.make_async_copy(
                view3.at[pl.ds(0, LANES)], ttiles_v, sem_t).wait()
            tokv = lax.iota(jnp.int32, LANES)
            subv = jnp.bitwise_and(
                tidx_v[pl.ds(0, LANES)], jnp.full((LANES,), 7, jnp.int32))

            def textract(col, _):
                colv = jnp.full((LANES,), col, jnp.int32)
                v = plsc.load_gather(ttiles_v, [tokv, subv, colv])
                plsc.store_scatter(trowsT_v, [colv, tokv], v)
                return _

            lax.fori_loop(0, d_model, textract, 0)
            pltpu.sync_copy(trowsT_v, tail_hbm.at[b])

    return emb


def kernel(input_ids, embedding_table):
    batch, seq = input_ids.shape
    vocab, d_model = embedding_table.shape
    padded = jnp.pad(input_ids, ((0, 0), (1, 7)), constant_values=BOS)
    flat_ids = padded.reshape(-1).astype(jnp.int32)
    emb = _build(batch, seq, vocab, d_model)
    outT, tailT = emb(flat_ids, embedding_table)
    out = outT.transpose(0, 2, 1)
    tail = tailT[:, :, 8]
    return out.at[:, seq, :].set(tail)
